# baseline (speedup 1.0000x reference)
import functools

import jax
import jax.numpy as jnp
from jax import lax
from jax.experimental import pallas as pl
from jax.experimental.pallas import tpu as pltpu
from jax.experimental.pallas import tpu_sc as plsc

T, H, I, E, K = 2048, 768, 3072, 8, 2
A = T * K
MAXH = 500.0
EPS = 1e-4
TILE = 128
R_PAD = A + E * TILE
NT = R_PAD // TILE
IB = 1536
NI = I // IB
ST = 256
NW = 32

_tc_call = pl.pallas_call


def _sc_mesh():
    return plsc.VectorSubcoreMesh(core_axis_name="c", subcore_axis_name="s")


def _clamp(x, m=MAXH):
    x = jnp.nan_to_num(x, nan=0.0, posinf=m, neginf=-m)
    return jnp.clip(x, -m, m)


def _silu(x):
    return x * (1.0 / (1.0 + jnp.exp(-x)))


def _router_body(x_ref, g_ref, b_ref, rw_ref, i0_ref, i1_ref, w0_ref, w1_ref,
                 aux_ref):
    x = _clamp(x_ref[...])
    mu = jnp.mean(x, axis=1, keepdims=True)
    var = jnp.mean((x - mu) ** 2, axis=1, keepdims=True)
    xn = (x - mu) * lax.rsqrt(var + 1e-5) * g_ref[...][None, :] + b_ref[...][None, :]
    xn = _clamp(xn, 50.0)
    logits = lax.dot_general(xn, rw_ref[...], (((1,), (1,)), ((), ())),
                             preferred_element_type=jnp.float32)
    logits = jnp.clip(logits, -10.0, 10.0)
    m = jnp.max(logits, axis=1, keepdims=True)
    ex = jnp.exp(logits - m)
    probs = jnp.clip(ex / jnp.sum(ex, axis=1, keepdims=True), EPS, 1.0)
    ids = lax.broadcasted_iota(jnp.int32, (T, E), 1)
    p0 = jnp.max(probs, axis=1)
    i0 = jnp.argmax(probs, axis=1).astype(jnp.int32)
    masked = jnp.where(ids == i0[:, None], -1.0, probs)
    p1 = jnp.max(masked, axis=1)
    i1 = jnp.argmax(masked, axis=1).astype(jnp.int32)
    s = jnp.clip(p0 + p1, EPS, None)
    i0_ref[...] = i0
    i1_ref[...] = i1
    w0_ref[...] = p0 / s
    w1_ref[...] = p1 / s
    onehot = ((ids == i0[:, None]) | (ids == i1[:, None])).astype(jnp.float32)
    frac = jnp.mean(onehot, axis=0)
    mean_prob = jnp.mean(probs, axis=0)
    aux_ref[...] = jnp.reshape((E / K) * jnp.sum(frac * mean_prob), (1, 1))


def _router(xflat, ln_g, ln_b, rw):
    return _tc_call(
        _router_body,
        out_shape=[
            jax.ShapeDtypeStruct((T,), jnp.int32),
            jax.ShapeDtypeStruct((T,), jnp.int32),
            jax.ShapeDtypeStruct((T,), jnp.float32),
            jax.ShapeDtypeStruct((T,), jnp.float32),
            jax.ShapeDtypeStruct((1, 1), jnp.float32),
        ],
    )(xflat, ln_g, ln_b, rw)


def _sc_gather(src, idx):
    per_w = R_PAD // NW
    chunk = per_w // 2

    @functools.partial(
        pl.kernel,
        out_type=jax.ShapeDtypeStruct((R_PAD, H), jnp.float32),
        mesh=_sc_mesh(),
        scratch_types=[
            pltpu.VMEM((chunk,), jnp.int32),
            pltpu.VMEM((chunk, H), jnp.float32),
            pltpu.SemaphoreType.DMA,
        ],
    )
    def k(src_hbm, idx_hbm, out_hbm, idx_v, rows_v, sem):
        wid = lax.axis_index("s") * 2 + lax.axis_index("c")
        for c in range(2):
            base = wid * per_w + c * chunk
            pltpu.sync_copy(idx_hbm.at[pl.ds(base, chunk)], idx_v)
            pltpu.async_copy(src_hbm.at[idx_v], rows_v, sem).wait()
            pltpu.sync_copy(rows_v, out_hbm.at[pl.ds(base, chunk)])

    return k(src, idx)


def _c1_body(es_ref, xg_ref, gw_ref, uw_ref, prod_ref):
    x = _clamp(xg_ref[...])
    g = lax.dot_general(x, gw_ref[0], (((1,), (1,)), ((), ())),
                        preferred_element_type=jnp.float32)
    g = _silu(_clamp(g))
    u = _clamp(lax.dot_general(x, uw_ref[0], (((1,), (1,)), ((), ())),
                               preferred_element_type=jnp.float32))
    prod_ref[...] = jnp.clip(g * u, -MAXH, MAXH)


def _gateup(xg, gw, uw, tile_expert):
    return _tc_call(
        _c1_body,
        grid_spec=pltpu.PrefetchScalarGridSpec(
            num_scalar_prefetch=1,
            grid=(NI, NT),
            in_specs=[
                pl.BlockSpec((TILE, H), lambda i, j, es: (j, 0)),
                pl.BlockSpec((1, IB, H), lambda i, j, es: (es[j], i, 0)),
                pl.BlockSpec((1, IB, H), lambda i, j, es: (es[j], i, 0)),
            ],
            out_specs=pl.BlockSpec((TILE, IB), lambda i, j, es: (j, i)),
        ),
        out_shape=jax.ShapeDtypeStruct((R_PAD, I), jnp.float32),
    )(tile_expert, xg, gw, uw)


def _c2_body(es_ref, prod_ref, dw_ref, wr_ref, out_ref):
    o = lax.dot_general(prod_ref[...], dw_ref[0], (((1,), (1,)), ((), ())),
                        preferred_element_type=jnp.float32)
    out_ref[...] = _clamp(o) * wr_ref[...]


def _down(prod, dw, w_row, tile_expert):
    return _tc_call(
        _c2_body,
        grid_spec=pltpu.PrefetchScalarGridSpec(
            num_scalar_prefetch=1,
            grid=(NT,),
            in_specs=[
                pl.BlockSpec((TILE, I), lambda j, es: (j, 0)),
                pl.BlockSpec((1, H, I), lambda j, es: (es[j], 0, 0)),
                pl.BlockSpec((TILE, 1), lambda j, es: (j, 0)),
            ],
            out_specs=pl.BlockSpec((TILE, H), lambda j, es: (j, 0)),
        ),
        out_shape=jax.ShapeDtypeStruct((R_PAD, H), jnp.float32),
    )(tile_expert, prod, dw, w_row)


def _shared_body(x_ref, gw_ref, uw_ref, dw_ref, sg_ref, out_ref):
    x = _clamp(x_ref[...])
    g = _silu(_clamp(lax.dot_general(x, gw_ref[...], (((1,), (1,)), ((), ())),
                                     preferred_element_type=jnp.float32)))
    u = _clamp(lax.dot_general(x, uw_ref[...], (((1,), (1,)), ((), ())),
                               preferred_element_type=jnp.float32))
    p = jnp.clip(g * u, -MAXH, MAXH)
    o = _clamp(lax.dot_general(p, dw_ref[...], (((1,), (1,)), ((), ())),
                               preferred_element_type=jnp.float32))
    out_ref[...] = o * sg_ref[0]


def _shared(xflat, gw, uw, dw, sig):
    return _tc_call(
        _shared_body,
        grid=(T // ST,),
        in_specs=[
            pl.BlockSpec((ST, H), lambda t: (t, 0)),
            pl.BlockSpec((I, H), lambda t: (0, 0)),
            pl.BlockSpec((I, H), lambda t: (0, 0)),
            pl.BlockSpec((H, I), lambda t: (0, 0)),
            pl.BlockSpec(memory_space=pltpu.SMEM),
        ],
        out_specs=pl.BlockSpec((ST, H), lambda t: (t, 0)),
        out_shape=jax.ShapeDtypeStruct((T, H), jnp.float32),
    )(xflat, gw, uw, dw, sig)


def _sc_combine(eoutw, pos, shared):
    per_w = T // NW
    chunk = per_w // 2

    @functools.partial(
        pl.kernel,
        out_type=jax.ShapeDtypeStruct((T, H), jnp.float32),
        mesh=_sc_mesh(),
        scratch_types=[
            pltpu.VMEM((chunk,), jnp.int32),
            pltpu.VMEM((chunk, H), jnp.float32),
            pltpu.VMEM((chunk, H), jnp.float32),
            pltpu.VMEM((chunk, H), jnp.float32),
            pltpu.SemaphoreType.DMA,
        ],
    )
    def k(eoutw_hbm, pos_hbm, shared_hbm, out_hbm, idx_v, a_v, b_v, s_v, sem):
        wid = lax.axis_index("s") * 2 + lax.axis_index("c")
        for c in range(2):
            base = wid * per_w + c * chunk
            pltpu.sync_copy(pos_hbm.at[0, pl.ds(base, chunk)], idx_v)
            pltpu.async_copy(eoutw_hbm.at[idx_v], a_v, sem).wait()
            pltpu.sync_copy(pos_hbm.at[1, pl.ds(base, chunk)], idx_v)
            pltpu.async_copy(eoutw_hbm.at[idx_v], b_v, sem).wait()
            pltpu.sync_copy(shared_hbm.at[pl.ds(base, chunk)], s_v)

            def row_body(r, carry):
                def col_body(q, carry2):
                    sl = pl.ds(q * 16, 16)
                    v = a_v[r, sl] + b_v[r, sl] + s_v[r, sl]
                    s_v[r, sl] = jnp.clip(v, -MAXH, MAXH)
                    return carry2
                return lax.fori_loop(0, H // 16, col_body, carry)

            lax.fori_loop(0, chunk, row_body, 0)
            pltpu.sync_copy(s_v, out_hbm.at[pl.ds(base, chunk)])

    return k(eoutw, pos, shared)


def kernel(hidden_states, ln_gamma, ln_beta, router_w, expert_gate_w,
           expert_up_w, expert_down_w, shared_gate_w, shared_up_w,
           shared_down_w, shared_gate_param):
    b, s, h = hidden_states.shape
    xflat = hidden_states.reshape(T, H)

    i0, i1, w0, w1, aux = _router(xflat, ln_gamma, ln_beta, router_w)

    ef = jnp.stack([i0, i1], axis=1).reshape(-1)
    wf = jnp.stack([w0, w1], axis=1).reshape(-1)
    perm = jnp.argsort(ef)
    ef_s = ef[perm]
    counts = jnp.sum((ef[:, None] == jnp.arange(E)[None, :]).astype(jnp.int32),
                     axis=0)
    padded = ((counts + TILE - 1) // TILE) * TILE
    pstart = jnp.concatenate([jnp.zeros(1, jnp.int32),
                              jnp.cumsum(padded)[:-1].astype(jnp.int32)])
    gstart = jnp.concatenate([jnp.zeros(1, jnp.int32),
                              jnp.cumsum(counts)[:-1].astype(jnp.int32)])
    jidx = jnp.arange(A, dtype=jnp.int32)
    row_j = pstart[ef_s] + (jidx - gstart[ef_s])
    tok_row = jnp.zeros(R_PAD, jnp.int32).at[row_j].set(
        (perm // K).astype(jnp.int32))
    w_row = jnp.zeros(R_PAD, jnp.float32).at[row_j].set(wf[perm])
    pos = jnp.zeros(A, jnp.int32).at[perm].set(row_j.astype(jnp.int32))
    pos2 = pos.reshape(T, K).T
    tile_expert = jnp.minimum(
        jnp.searchsorted(jnp.cumsum(padded), jnp.arange(NT) * TILE,
                         side="right"),
        E - 1).astype(jnp.int32)

    xg = _sc_gather(xflat, tok_row)
    prod = _gateup(xg, expert_gate_w, expert_up_w, tile_expert)
    eoutw = _down(prod, expert_down_w, w_row.reshape(R_PAD, 1), tile_expert)

    sig = jax.nn.sigmoid(shared_gate_param)
    shared_out = _shared(xflat, shared_gate_w, shared_up_w, shared_down_w, sig)

    final = _sc_combine(eoutw, pos2, shared_out)
    return final.reshape(b, s, h), aux.reshape(())


# speedup vs baseline: 1.1156x; 1.1156x over previous
import functools

import jax
import jax.numpy as jnp
from jax import lax
from jax.experimental import pallas as pl
from jax.experimental.pallas import tpu as pltpu
from jax.experimental.pallas import tpu_sc as plsc

T, H, I, E, K = 2048, 768, 3072, 8, 2
A = T * K
MAXH = 500.0
EPS = 1e-4
TILE = 128
R_PAD = A + E * TILE
NT = R_PAD // TILE
IB = 1536
NI = I // IB
ST = 256
NW = 32

_tc_call = pl.pallas_call


def _sc_mesh():
    return plsc.VectorSubcoreMesh(core_axis_name="c", subcore_axis_name="s")


def _clamp(x, m=MAXH):
    x = jnp.nan_to_num(x, nan=0.0, posinf=m, neginf=-m)
    return jnp.clip(x, -m, m)


def _silu(x):
    return x * (1.0 / (1.0 + jnp.exp(-x)))


def _router_body(x_ref, g_ref, b_ref, rw_ref, i0_ref, i1_ref, w0_ref, w1_ref,
                 aux_ref):
    x = _clamp(x_ref[...])
    mu = jnp.mean(x, axis=1, keepdims=True)
    var = jnp.mean((x - mu) ** 2, axis=1, keepdims=True)
    xn = (x - mu) * lax.rsqrt(var + 1e-5) * g_ref[...][None, :] + b_ref[...][None, :]
    xn = _clamp(xn, 50.0)
    logits = lax.dot_general(xn, rw_ref[...], (((1,), (1,)), ((), ())),
                             preferred_element_type=jnp.float32)
    logits = jnp.clip(logits, -10.0, 10.0)
    m = jnp.max(logits, axis=1, keepdims=True)
    ex = jnp.exp(logits - m)
    probs = jnp.clip(ex / jnp.sum(ex, axis=1, keepdims=True), EPS, 1.0)
    ids = lax.broadcasted_iota(jnp.int32, (T, E), 1)
    p0 = jnp.max(probs, axis=1)
    i0 = jnp.argmax(probs, axis=1).astype(jnp.int32)
    masked = jnp.where(ids == i0[:, None], -1.0, probs)
    p1 = jnp.max(masked, axis=1)
    i1 = jnp.argmax(masked, axis=1).astype(jnp.int32)
    s = jnp.clip(p0 + p1, EPS, None)
    i0_ref[...] = i0
    i1_ref[...] = i1
    w0_ref[...] = p0 / s
    w1_ref[...] = p1 / s
    onehot = ((ids == i0[:, None]) | (ids == i1[:, None])).astype(jnp.float32)
    frac = jnp.mean(onehot, axis=0)
    mean_prob = jnp.mean(probs, axis=0)
    aux_ref[...] = jnp.reshape((E / K) * jnp.sum(frac * mean_prob), (1, 1))


def _router(xflat, ln_g, ln_b, rw):
    return _tc_call(
        _router_body,
        out_shape=[
            jax.ShapeDtypeStruct((T,), jnp.int32),
            jax.ShapeDtypeStruct((T,), jnp.int32),
            jax.ShapeDtypeStruct((T,), jnp.float32),
            jax.ShapeDtypeStruct((T,), jnp.float32),
            jax.ShapeDtypeStruct((1, 1), jnp.float32),
        ],
    )(xflat, ln_g, ln_b, rw)


def _sc_gather(src, idx):
    per_w = R_PAD // NW
    chunk = per_w // 2

    @functools.partial(
        pl.kernel,
        out_type=jax.ShapeDtypeStruct((R_PAD, H), jnp.float32),
        mesh=_sc_mesh(),
        scratch_types=[
            pltpu.VMEM((chunk,), jnp.int32),
            pltpu.VMEM((chunk,), jnp.int32),
            pltpu.VMEM((chunk, H), jnp.float32),
            pltpu.VMEM((chunk, H), jnp.float32),
            pltpu.SemaphoreType.DMA,
            pltpu.SemaphoreType.DMA,
            pltpu.SemaphoreType.DMA,
            pltpu.SemaphoreType.DMA,
        ],
    )
    def k(src_hbm, idx_hbm, out_hbm, idx_v, idx2_v, rows_v, rows2_v, s0, s1,
          s2, s3):
        wid = lax.axis_index("s") * 2 + lax.axis_index("c")
        b0 = wid * per_w
        b1 = b0 + chunk
        pltpu.sync_copy(idx_hbm.at[pl.ds(b0, chunk)], idx_v)
        r0 = pltpu.async_copy(src_hbm.at[idx_v], rows_v, s0)
        pltpu.sync_copy(idx_hbm.at[pl.ds(b1, chunk)], idx2_v)
        r1 = pltpu.async_copy(src_hbm.at[idx2_v], rows2_v, s1)
        r0.wait()
        w0 = pltpu.async_copy(rows_v, out_hbm.at[pl.ds(b0, chunk)], s2)
        r1.wait()
        w1 = pltpu.async_copy(rows2_v, out_hbm.at[pl.ds(b1, chunk)], s3)
        w0.wait()
        w1.wait()

    return k(src, idx)


def _ffn_body(es_ref, xg_ref, gw_ref, uw_ref, dw_ref, wr_ref, out_ref):
    x = _clamp(xg_ref[...])
    g = _silu(_clamp(lax.dot_general(x, gw_ref[0], (((1,), (1,)), ((), ())),
                                     preferred_element_type=jnp.float32)))
    u = _clamp(lax.dot_general(x, uw_ref[0], (((1,), (1,)), ((), ())),
                               preferred_element_type=jnp.float32))
    p = jnp.clip(g * u, -MAXH, MAXH)
    o = lax.dot_general(p, dw_ref[0], (((1,), (1,)), ((), ())),
                        preferred_element_type=jnp.float32)
    out_ref[...] = _clamp(o) * wr_ref[...]


def _ffn(xg, gw, uw, dw, w_row, tile_expert):
    return _tc_call(
        _ffn_body,
        grid_spec=pltpu.PrefetchScalarGridSpec(
            num_scalar_prefetch=1,
            grid=(NT,),
            in_specs=[
                pl.BlockSpec((TILE, H), lambda j, es: (j, 0)),
                pl.BlockSpec((1, I, H), lambda j, es: (es[j], 0, 0)),
                pl.BlockSpec((1, I, H), lambda j, es: (es[j], 0, 0)),
                pl.BlockSpec((1, H, I), lambda j, es: (es[j], 0, 0)),
                pl.BlockSpec((TILE, 1), lambda j, es: (j, 0)),
            ],
            out_specs=pl.BlockSpec((TILE, H), lambda j, es: (j, 0)),
        ),
        out_shape=jax.ShapeDtypeStruct((R_PAD, H), jnp.float32),
    )(tile_expert, xg, gw, uw, dw, w_row)


def _shared_body(x_ref, gw_ref, uw_ref, dw_ref, sg_ref, out_ref):
    x = _clamp(x_ref[...])
    g = _silu(_clamp(lax.dot_general(x, gw_ref[...], (((1,), (1,)), ((), ())),
                                     preferred_element_type=jnp.float32)))
    u = _clamp(lax.dot_general(x, uw_ref[...], (((1,), (1,)), ((), ())),
                               preferred_element_type=jnp.float32))
    p = jnp.clip(g * u, -MAXH, MAXH)
    o = _clamp(lax.dot_general(p, dw_ref[...], (((1,), (1,)), ((), ())),
                               preferred_element_type=jnp.float32))
    out_ref[...] = o * sg_ref[0]


def _shared(xflat, gw, uw, dw, sig):
    return _tc_call(
        _shared_body,
        grid=(T // ST,),
        in_specs=[
            pl.BlockSpec((ST, H), lambda t: (t, 0)),
            pl.BlockSpec((I, H), lambda t: (0, 0)),
            pl.BlockSpec((I, H), lambda t: (0, 0)),
            pl.BlockSpec((H, I), lambda t: (0, 0)),
            pl.BlockSpec(memory_space=pltpu.SMEM),
        ],
        out_specs=pl.BlockSpec((ST, H), lambda t: (t, 0)),
        out_shape=jax.ShapeDtypeStruct((T, H), jnp.float32),
    )(xflat, gw, uw, dw, sig)


def _sc_combine(eoutw, pos, shared):
    per_w = T // NW
    chunk = per_w // 2

    @functools.partial(
        pl.kernel,
        out_type=jax.ShapeDtypeStruct((T, H), jnp.float32),
        mesh=_sc_mesh(),
        scratch_types=[
            pltpu.VMEM((chunk,), jnp.int32),
            pltpu.VMEM((chunk,), jnp.int32),
            pltpu.VMEM((chunk, H), jnp.float32),
            pltpu.VMEM((chunk, H), jnp.float32),
            pltpu.VMEM((chunk, H), jnp.float32),
            pltpu.VMEM((chunk, H), jnp.float32),
            pltpu.SemaphoreType.DMA,
            pltpu.SemaphoreType.DMA,
            pltpu.SemaphoreType.DMA,
            pltpu.SemaphoreType.DMA,
        ],
    )
    def k(eoutw_hbm, pos_hbm, shared_hbm, out_hbm, ia_v, ib_v, a_v, b_v,
          s0_v, s1_v, sem_a, sem_b, sem_s, sem_w):
        wid = lax.axis_index("s") * 2 + lax.axis_index("c")
        wq = []
        for c in range(2):
            base = wid * per_w + c * chunk
            s_v = s0_v if c == 0 else s1_v
            pltpu.sync_copy(pos_hbm.at[0, pl.ds(base, chunk)], ia_v)
            ga = pltpu.async_copy(eoutw_hbm.at[ia_v], a_v, sem_a)
            pltpu.sync_copy(pos_hbm.at[1, pl.ds(base, chunk)], ib_v)
            gb = pltpu.async_copy(eoutw_hbm.at[ib_v], b_v, sem_b)
            gs = pltpu.async_copy(shared_hbm.at[pl.ds(base, chunk)], s_v,
                                  sem_s)
            ga.wait()
            gb.wait()
            gs.wait()

            def row_body(r, carry):
                def col_body(q, carry2):
                    for u in range(4):
                        sl = pl.ds((q * 4 + u) * 16, 16)
                        v = a_v[r, sl] + b_v[r, sl] + s_v[r, sl]
                        s_v[r, sl] = jnp.clip(v, -MAXH, MAXH)
                    return carry2
                return lax.fori_loop(0, H // 64, col_body, carry)

            lax.fori_loop(0, chunk, row_body, 0)
            wq.append(
                pltpu.async_copy(s_v, out_hbm.at[pl.ds(base, chunk)], sem_w))
        for w in wq:
            w.wait()

    return k(eoutw, pos, shared)


def kernel(hidden_states, ln_gamma, ln_beta, router_w, expert_gate_w,
           expert_up_w, expert_down_w, shared_gate_w, shared_up_w,
           shared_down_w, shared_gate_param):
    b, s, h = hidden_states.shape
    xflat = hidden_states.reshape(T, H)

    i0, i1, w0, w1, aux = _router(xflat, ln_gamma, ln_beta, router_w)

    ef = jnp.stack([i0, i1], axis=1).reshape(-1)
    wf = jnp.stack([w0, w1], axis=1).reshape(-1)
    perm = jnp.argsort(ef)
    ef_s = ef[perm]
    counts = jnp.sum((ef[:, None] == jnp.arange(E)[None, :]).astype(jnp.int32),
                     axis=0)
    padded = ((counts + TILE - 1) // TILE) * TILE
    pstart = jnp.concatenate([jnp.zeros(1, jnp.int32),
                              jnp.cumsum(padded)[:-1].astype(jnp.int32)])
    gstart = jnp.concatenate([jnp.zeros(1, jnp.int32),
                              jnp.cumsum(counts)[:-1].astype(jnp.int32)])
    jidx = jnp.arange(A, dtype=jnp.int32)
    row_j = pstart[ef_s] + (jidx - gstart[ef_s])
    tok_row = jnp.zeros(R_PAD, jnp.int32).at[row_j].set(
        (perm // K).astype(jnp.int32))
    w_row = jnp.zeros(R_PAD, jnp.float32).at[row_j].set(wf[perm])
    pos = jnp.zeros(A, jnp.int32).at[perm].set(row_j.astype(jnp.int32))
    pos2 = pos.reshape(T, K).T
    tile_expert = jnp.minimum(
        jnp.searchsorted(jnp.cumsum(padded), jnp.arange(NT) * TILE,
                         side="right"),
        E - 1).astype(jnp.int32)

    xg = _sc_gather(xflat, tok_row)
    eoutw = _ffn(xg, expert_gate_w, expert_up_w, expert_down_w,
                 w_row.reshape(R_PAD, 1), tile_expert)

    sig = jax.nn.sigmoid(shared_gate_param)
    shared_out = _shared(xflat, shared_gate_w, shared_up_w, shared_down_w, sig)

    final = _sc_combine(eoutw, pos2, shared_out)
    return final.reshape(b, s, h), aux.reshape(())


# speedup vs baseline: 1.1506x; 1.0314x over previous
import functools

import jax
import jax.numpy as jnp
from jax import lax
from jax.experimental import pallas as pl
from jax.experimental.pallas import tpu as pltpu
from jax.experimental.pallas import tpu_sc as plsc

T, H, I, E, K = 2048, 768, 3072, 8, 2
A = T * K
MAXH = 500.0
EPS = 1e-4
TILE = 128
R_PAD = A + E * TILE
NT = R_PAD // TILE
IB = 1536
NI = I // IB
ST = 256
NW = 32

_tc_call = pl.pallas_call


def _sc_mesh():
    return plsc.VectorSubcoreMesh(core_axis_name="c", subcore_axis_name="s")


def _clamp(x, m=MAXH):
    x = jnp.nan_to_num(x, nan=0.0, posinf=m, neginf=-m)
    return jnp.clip(x, -m, m)


def _silu(x):
    return x * (1.0 / (1.0 + jnp.exp(-x)))


def _router_body(x_ref, g_ref, b_ref, rw_ref, i0_ref, i1_ref, w0_ref, w1_ref,
                 aux_ref):
    x = _clamp(x_ref[...])
    mu = jnp.mean(x, axis=1, keepdims=True)
    var = jnp.mean((x - mu) ** 2, axis=1, keepdims=True)
    xn = (x - mu) * lax.rsqrt(var + 1e-5) * g_ref[...][None, :] + b_ref[...][None, :]
    xn = _clamp(xn, 50.0)
    logits = lax.dot_general(xn, rw_ref[...], (((1,), (1,)), ((), ())),
                             preferred_element_type=jnp.float32)
    logits = jnp.clip(logits, -10.0, 10.0)
    m = jnp.max(logits, axis=1, keepdims=True)
    ex = jnp.exp(logits - m)
    probs = jnp.clip(ex / jnp.sum(ex, axis=1, keepdims=True), EPS, 1.0)
    ids = lax.broadcasted_iota(jnp.int32, (T, E), 1)
    p0 = jnp.max(probs, axis=1)
    i0 = jnp.argmax(probs, axis=1).astype(jnp.int32)
    masked = jnp.where(ids == i0[:, None], -1.0, probs)
    p1 = jnp.max(masked, axis=1)
    i1 = jnp.argmax(masked, axis=1).astype(jnp.int32)
    s = jnp.clip(p0 + p1, EPS, None)
    i0_ref[...] = i0
    i1_ref[...] = i1
    w0_ref[...] = p0 / s
    w1_ref[...] = p1 / s
    onehot = ((ids == i0[:, None]) | (ids == i1[:, None])).astype(jnp.float32)
    frac = jnp.mean(onehot, axis=0)
    mean_prob = jnp.mean(probs, axis=0)
    aux_ref[...] = jnp.reshape((E / K) * jnp.sum(frac * mean_prob), (1, 1))


def _router(xflat, ln_g, ln_b, rw):
    return _tc_call(
        _router_body,
        out_shape=[
            jax.ShapeDtypeStruct((T,), jnp.int32),
            jax.ShapeDtypeStruct((T,), jnp.int32),
            jax.ShapeDtypeStruct((T,), jnp.float32),
            jax.ShapeDtypeStruct((T,), jnp.float32),
            jax.ShapeDtypeStruct((1, 1), jnp.float32),
        ],
    )(xflat, ln_g, ln_b, rw)


def _ffn_body(es_ref, trow_ref, x_ref, gw_ref, uw_ref, dw_ref, wr_ref,
              out_ref, xg_s):
    j = pl.program_id(0)

    def gath(r, c):
        t = trow_ref[j * TILE + r]
        xg_s[pl.ds(r, 1), :] = x_ref[pl.ds(t, 1), :]
        return c

    lax.fori_loop(0, TILE, gath, 0)
    x = _clamp(xg_s[...])
    g = _silu(_clamp(lax.dot_general(x, gw_ref[0], (((1,), (1,)), ((), ())),
                                     preferred_element_type=jnp.float32)))
    u = _clamp(lax.dot_general(x, uw_ref[0], (((1,), (1,)), ((), ())),
                               preferred_element_type=jnp.float32))
    p = jnp.clip(g * u, -MAXH, MAXH)
    o = lax.dot_general(p, dw_ref[0], (((1,), (1,)), ((), ())),
                        preferred_element_type=jnp.float32)
    out_ref[...] = _clamp(o) * wr_ref[...]


def _ffn(xflat, gw, uw, dw, w_row, tile_expert, tok_row):
    return _tc_call(
        _ffn_body,
        grid_spec=pltpu.PrefetchScalarGridSpec(
            num_scalar_prefetch=2,
            grid=(NT,),
            in_specs=[
                pl.BlockSpec((T, H), lambda j, es, tr: (0, 0)),
                pl.BlockSpec((1, I, H), lambda j, es, tr: (es[j], 0, 0)),
                pl.BlockSpec((1, I, H), lambda j, es, tr: (es[j], 0, 0)),
                pl.BlockSpec((1, H, I), lambda j, es, tr: (es[j], 0, 0)),
                pl.BlockSpec((TILE, 1), lambda j, es, tr: (j, 0)),
            ],
            out_specs=pl.BlockSpec((TILE, H), lambda j, es, tr: (j, 0)),
            scratch_shapes=[pltpu.VMEM((TILE, H), jnp.float32)],
        ),
        compiler_params=pltpu.CompilerParams(
            vmem_limit_bytes=128 * 1024 * 1024),
        out_shape=jax.ShapeDtypeStruct((R_PAD, H), jnp.float32),
    )(tile_expert, tok_row, xflat, gw, uw, dw, w_row)


def _shared_body(x_ref, gw_ref, uw_ref, dw_ref, sg_ref, out_ref):
    x = _clamp(x_ref[...])
    g = _silu(_clamp(lax.dot_general(x, gw_ref[...], (((1,), (1,)), ((), ())),
                                     preferred_element_type=jnp.float32)))
    u = _clamp(lax.dot_general(x, uw_ref[...], (((1,), (1,)), ((), ())),
                               preferred_element_type=jnp.float32))
    p = jnp.clip(g * u, -MAXH, MAXH)
    o = _clamp(lax.dot_general(p, dw_ref[...], (((1,), (1,)), ((), ())),
                               preferred_element_type=jnp.float32))
    out_ref[...] = o * sg_ref[0]


def _shared(xflat, gw, uw, dw, sig):
    return _tc_call(
        _shared_body,
        grid=(T // ST,),
        in_specs=[
            pl.BlockSpec((ST, H), lambda t: (t, 0)),
            pl.BlockSpec((I, H), lambda t: (0, 0)),
            pl.BlockSpec((I, H), lambda t: (0, 0)),
            pl.BlockSpec((H, I), lambda t: (0, 0)),
            pl.BlockSpec(memory_space=pltpu.SMEM),
        ],
        out_specs=pl.BlockSpec((ST, H), lambda t: (t, 0)),
        out_shape=jax.ShapeDtypeStruct((T, H), jnp.float32),
    )(xflat, gw, uw, dw, sig)


def _sc_combine(eoutw, pos, shared):
    per_w = T // NW
    chunk = per_w // 2

    @functools.partial(
        pl.kernel,
        out_type=jax.ShapeDtypeStruct((T, H), jnp.float32),
        mesh=_sc_mesh(),
        scratch_types=[
            pltpu.VMEM((chunk,), jnp.int32),
            pltpu.VMEM((chunk,), jnp.int32),
            pltpu.VMEM((chunk, H), jnp.float32),
            pltpu.VMEM((chunk, H), jnp.float32),
            pltpu.VMEM((chunk, H), jnp.float32),
            pltpu.VMEM((chunk, H), jnp.float32),
            pltpu.SemaphoreType.DMA,
            pltpu.SemaphoreType.DMA,
            pltpu.SemaphoreType.DMA,
            pltpu.SemaphoreType.DMA,
        ],
    )
    def k(eoutw_hbm, pos_hbm, shared_hbm, out_hbm, ia_v, ib_v, a_v, b_v,
          s0_v, s1_v, sem_a, sem_b, sem_s, sem_w):
        wid = lax.axis_index("s") * 2 + lax.axis_index("c")
        wq = []
        for c in range(2):
            base = wid * per_w + c * chunk
            s_v = s0_v if c == 0 else s1_v
            pltpu.sync_copy(pos_hbm.at[0, pl.ds(base, chunk)], ia_v)
            ga = pltpu.async_copy(eoutw_hbm.at[ia_v], a_v, sem_a)
            pltpu.sync_copy(pos_hbm.at[1, pl.ds(base, chunk)], ib_v)
            gb = pltpu.async_copy(eoutw_hbm.at[ib_v], b_v, sem_b)
            gs = pltpu.async_copy(shared_hbm.at[pl.ds(base, chunk)], s_v,
                                  sem_s)
            ga.wait()
            gb.wait()
            gs.wait()

            def row_body(r, carry):
                def col_body(q, carry2):
                    for u in range(4):
                        sl = pl.ds((q * 4 + u) * 16, 16)
                        v = a_v[r, sl] + b_v[r, sl] + s_v[r, sl]
                        s_v[r, sl] = jnp.clip(v, -MAXH, MAXH)
                    return carry2
                return lax.fori_loop(0, H // 64, col_body, carry)

            lax.fori_loop(0, chunk, row_body, 0)
            wq.append(
                pltpu.async_copy(s_v, out_hbm.at[pl.ds(base, chunk)], sem_w))
        for w in wq:
            w.wait()

    return k(eoutw, pos, shared)


def kernel(hidden_states, ln_gamma, ln_beta, router_w, expert_gate_w,
           expert_up_w, expert_down_w, shared_gate_w, shared_up_w,
           shared_down_w, shared_gate_param):
    b, s, h = hidden_states.shape
    xflat = hidden_states.reshape(T, H)

    i0, i1, w0, w1, aux = _router(xflat, ln_gamma, ln_beta, router_w)

    ef = jnp.stack([i0, i1], axis=1).reshape(-1)
    wf = jnp.stack([w0, w1], axis=1).reshape(-1)
    onehot = (ef[:, None] == jnp.arange(E)[None, :]).astype(jnp.int32)
    rank = jnp.sum((jnp.cumsum(onehot, axis=0) - onehot) * onehot, axis=1)
    counts = jnp.sum(onehot, axis=0)
    padded = ((counts + TILE - 1) // TILE) * TILE
    pstart = jnp.concatenate([jnp.zeros(1, jnp.int32),
                              jnp.cumsum(padded)[:-1].astype(jnp.int32)])
    row_a = (pstart[ef] + rank).astype(jnp.int32)
    tok_row = jnp.zeros(R_PAD, jnp.int32).at[row_a].set(
        jnp.arange(A, dtype=jnp.int32) // K)
    w_row = jnp.zeros(R_PAD, jnp.float32).at[row_a].set(wf)
    pos2 = row_a.reshape(T, K).T
    tile_expert = jnp.minimum(
        jnp.searchsorted(jnp.cumsum(padded), jnp.arange(NT) * TILE,
                         side="right"),
        E - 1).astype(jnp.int32)

    eoutw = _ffn(xflat, expert_gate_w, expert_up_w, expert_down_w,
                 w_row.reshape(R_PAD, 1), tile_expert, tok_row)

    sig = jax.nn.sigmoid(shared_gate_param)
    shared_out = _shared(xflat, shared_gate_w, shared_up_w, shared_down_w, sig)

    final = _sc_combine(eoutw, pos2, shared_out)
    return final.reshape(b, s, h), aux.reshape(())
